# baseline (speedup 1.0000x reference)
import functools

import jax
import jax.numpy as jnp
from jax import lax
from jax.experimental import pallas as pl
from jax.experimental.pallas import tpu as pltpu
from jax.experimental.pallas import tpu_sc as plsc

IN_C = 384
OUT_C = 192
HW = 224 * 224


def _topk_body(p_row_ref, p_col_ref, vals_ref, idx_ref):
    p_row = p_row_ref[...]
    p_col = p_col_ref[...]
    gt = (p_row > p_col).astype(jnp.int32)
    jj = lax.broadcasted_iota(jnp.int32, (IN_C, IN_C), 1)
    ii = lax.broadcasted_iota(jnp.int32, (IN_C, IN_C), 0)
    tie = ((p_row == p_col) & (jj < ii)).astype(jnp.int32)
    rank = jnp.sum(gt + tie, axis=1, keepdims=True)
    rr = lax.broadcasted_iota(jnp.int32, (IN_C, OUT_C), 1)
    m = rank == rr
    vals_ref[...] = jnp.sum(
        jnp.where(m, p_col, jnp.float32(0)), axis=0, keepdims=True)
    ii_c = lax.broadcasted_iota(jnp.int32, (IN_C, OUT_C), 0)
    idx_ref[...] = jnp.sum(
        jnp.where(m, ii_c, 0), axis=0, keepdims=True)


def _topk(params):
    p_row = params.reshape(1, IN_C)
    p_col = params.reshape(IN_C, 1)
    vals, idx = pl.pallas_call(
        _topk_body,
        out_shape=(
            jax.ShapeDtypeStruct((1, OUT_C), jnp.float32),
            jax.ShapeDtypeStruct((1, OUT_C), jnp.int32),
        ),
    )(p_row, p_col)
    return vals.reshape(OUT_C), idx.reshape(OUT_C)


def _gather_body(idx_ref, w_ref, x_ref, o_ref):
    i = pl.program_id(0)
    w = w_ref[i % OUT_C]
    o_ref[...] = w * x_ref[...]


def _gather(x, idx, w):
    grid_spec = pltpu.PrefetchScalarGridSpec(
        num_scalar_prefetch=2,
        grid=(2 * OUT_C,),
        in_specs=[
            pl.BlockSpec(
                (1, 1, 224, 224),
                lambda i, idx_ref, w_ref: (
                    i // OUT_C, idx_ref[i % OUT_C], 0, 0),
            ),
        ],
        out_specs=pl.BlockSpec(
            (1, 1, 224, 224), lambda i, idx_ref, w_ref: (i // OUT_C, i % OUT_C, 0, 0)),
    )
    return pl.pallas_call(
        _gather_body,
        grid_spec=grid_spec,
        out_shape=jax.ShapeDtypeStruct((2, OUT_C, 224, 224), jnp.float32),
    )(idx, w, x)


NW = 32
SLABS_PW = 2 * 224 // NW
WBLK = 56
NQ = 224 // WBLK
NVEC = OUT_C // 16


def _sc_body(xt_hbm, idx_hbm, w_hbm, out_hbm,
             sel_idx_v, sel_w_v,
             ibuf0, ibuf1, obuf0, obuf1,
             g_sem0, g_sem1, s_sem0, s_sem1):
    wid = lax.axis_index("s") * 2 + lax.axis_index("c")
    b = wid >> 4
    h0 = (wid & 15) * SLABS_PW

    pltpu.sync_copy(idx_hbm, sel_idx_v)
    pltpu.sync_copy(w_hbm, sel_w_v)

    chs = [sel_idx_v[pl.ds(p * 16, 16)] for p in range(NVEC)]
    wvs = [sel_w_v[pl.ds(p * 16, 16)] for p in range(NVEC)]

    ibufs = (ibuf0, ibuf1)
    obufs = (obuf0, obuf1)
    g_sems = (g_sem0, g_sem1)
    s_sems = (s_sem0, s_sem1)

    def unit_hw(u):
        return h0 + (u >> 2), (u & 3) * WBLK

    def gather(u, k):
        h, w0 = unit_hw(u)
        return pltpu.async_copy(
            xt_hbm.at[b, h, pl.ds(w0, WBLK)], ibufs[k], g_sems[k])

    NU = SLABS_PW * NQ

    def scale_into(k):
        ib = ibufs[k]
        ob = obufs[k]

        def body(r, _):
            rv = jnp.full((16,), r, jnp.int32)
            for p in range(NVEC):
                v = plsc.load_gather(ib, [rv, chs[p]])
                ob[r, pl.ds(p * 16, 16)] = v * wvs[p]
            return 0

        lax.fori_loop(0, WBLK, body, 0)

    def do_unit(u, k):
        nxt = 1 - k

        @pl.when(u >= 1)
        def _():
            pltpu.make_async_copy(
                obufs[nxt], out_hbm.at[b, h0, pl.ds(0, WBLK)],
                s_sems[nxt]).wait()

        @pl.when(u + 1 < NU)
        def _():
            gather(u + 1, nxt)

        pltpu.make_async_copy(
            xt_hbm.at[b, h0, pl.ds(0, WBLK)], ibufs[k], g_sems[k]).wait()
        scale_into(k)
        h, w0 = unit_hw(u)
        pltpu.async_copy(
            obufs[k], out_hbm.at[b, h, pl.ds(w0, WBLK)], s_sems[k])

    gather(0, 0)

    def lbody(t, _):
        do_unit(2 * t, 0)
        do_unit(2 * t + 1, 1)
        return 0

    lax.fori_loop(0, NU // 2, lbody, 0)
    pltpu.make_async_copy(
        obufs[1], out_hbm.at[b, h0, pl.ds(0, WBLK)], s_sems[1]).wait()


def _sc_gather(xt, idx, w):
    mesh = plsc.VectorSubcoreMesh(core_axis_name="c", subcore_axis_name="s")
    f = pl.kernel(
        _sc_body,
        mesh=mesh,
        compiler_params=pltpu.CompilerParams(
            needs_layout_passes=False, use_tc_tiling_on_sc=True),
        out_type=jax.ShapeDtypeStruct((2, 224, 224, OUT_C), jnp.float32),
        scratch_types=[
            pltpu.VMEM((OUT_C,), jnp.int32),
            pltpu.VMEM((OUT_C,), jnp.float32),
            pltpu.VMEM((WBLK, IN_C), jnp.float32),
            pltpu.VMEM((WBLK, IN_C), jnp.float32),
            pltpu.VMEM((WBLK, OUT_C), jnp.float32),
            pltpu.VMEM((WBLK, OUT_C), jnp.float32),
            pltpu.SemaphoreType.DMA,
            pltpu.SemaphoreType.DMA,
            pltpu.SemaphoreType.DMA,
            pltpu.SemaphoreType.DMA,
        ],
    )
    return f(xt, idx, w)


@jax.jit
def kernel(x, params):
    w, idx = _topk(params)
    xt = jnp.transpose(x, (0, 2, 3, 1))
    out = _sc_gather(xt, idx, w)
    return jnp.transpose(out, (0, 3, 1, 2))


# speedup vs baseline: 3.1312x; 3.1312x over previous
import functools

import jax
import jax.numpy as jnp
from jax import lax
from jax.experimental import pallas as pl
from jax.experimental.pallas import tpu as pltpu
from jax.experimental.pallas import tpu_sc as plsc

IN_C = 384
OUT_C = 192
HW = 224 * 224


def _topk_body(p_row_ref, p_col_ref, vals_ref, idx_ref):
    p_row = p_row_ref[...]
    p_col = p_col_ref[...]
    gt = (p_row > p_col).astype(jnp.int32)
    jj = lax.broadcasted_iota(jnp.int32, (IN_C, IN_C), 1)
    ii = lax.broadcasted_iota(jnp.int32, (IN_C, IN_C), 0)
    tie = ((p_row == p_col) & (jj < ii)).astype(jnp.int32)
    rank = jnp.sum(gt + tie, axis=1, keepdims=True)
    rr = lax.broadcasted_iota(jnp.int32, (IN_C, OUT_C), 1)
    m = rank == rr
    vals_ref[...] = jnp.sum(
        jnp.where(m, p_col, jnp.float32(0)), axis=0, keepdims=True)
    ii_c = lax.broadcasted_iota(jnp.int32, (IN_C, OUT_C), 0)
    idx_ref[...] = jnp.sum(
        jnp.where(m, ii_c, 0), axis=0, keepdims=True)


def _topk(params):
    p_row = params.reshape(1, IN_C)
    p_col = params.reshape(IN_C, 1)
    vals, idx = pl.pallas_call(
        _topk_body,
        out_shape=(
            jax.ShapeDtypeStruct((1, OUT_C), jnp.float32),
            jax.ShapeDtypeStruct((1, OUT_C), jnp.int32),
        ),
    )(p_row, p_col)
    return vals.reshape(OUT_C), idx.reshape(OUT_C)


def _gather_body(idx_ref, w_ref, x_ref, o_ref):
    i = pl.program_id(0)
    w = w_ref[i % OUT_C]
    o_ref[...] = w * x_ref[...]


def _gather(x, idx, w):
    grid_spec = pltpu.PrefetchScalarGridSpec(
        num_scalar_prefetch=2,
        grid=(2 * OUT_C,),
        in_specs=[
            pl.BlockSpec(
                (1, 1, 224, 224),
                lambda i, idx_ref, w_ref: (
                    i // OUT_C, idx_ref[i % OUT_C], 0, 0),
            ),
        ],
        out_specs=pl.BlockSpec(
            (1, 1, 224, 224), lambda i, idx_ref, w_ref: (i // OUT_C, i % OUT_C, 0, 0)),
    )
    return pl.pallas_call(
        _gather_body,
        grid_spec=grid_spec,
        out_shape=jax.ShapeDtypeStruct((2, OUT_C, 224, 224), jnp.float32),
    )(idx, w, x)


NW = 32
NCH = 2 * OUT_C // NW
LIST_N = NCH * 8


def _sc_body(x_hbm, idx_hbm, w_hbm, out_hbm,
             sel_idx_v, sel_w_v, idx_list_v, w_list_v,
             buf0, buf1, g_sem0, g_sem1, s_sem0, s_sem1):
    wid = lax.axis_index("s") * 2 + lax.axis_index("c")
    half = wid >> 4
    jbase = (wid & 15) * NCH

    pltpu.sync_copy(idx_hbm, sel_idx_v)
    pltpu.sync_copy(w_hbm, sel_w_v)

    lanes = lax.iota(jnp.int32, 16)
    base_v = jnp.full((16,), jbase * 8, jnp.int32)
    c_outc = jnp.full((16,), OUT_C, jnp.int32)
    for k in range(LIST_N // 16):
        rvec = base_v + jnp.full((16,), k * 16, jnp.int32) + lanes
        j = (rvec >> 3) % c_outc
        ch = plsc.load_gather(sel_idx_v, [j])
        idx_list_v[pl.ds(k * 16, 16)] = jnp.full(
            (16,), half * IN_C, jnp.int32) + ch
        w_list_v[pl.ds(k * 16, 16)] = plsc.load_gather(sel_w_v, [j])

    bufs = (buf0, buf1)
    g_sems = (g_sem0, g_sem1)
    s_sems = (s_sem0, s_sem1)

    def gather(g):
        src = idx_list_v[pl.ds(g * 8, 16)][0]
        return pltpu.async_copy(
            x_hbm.at[pl.ds(src, 1)], bufs[g % 2], g_sems[g % 2])

    def scale_channel(bb, g):
        ws = jnp.full((16,), w_list_v[pl.ds(g * 8, 16)][0], jnp.float32)

        def body(h, _):
            for o in range(0, 224, 16):
                bb[0, h, pl.ds(o, 16)] = bb[0, h, pl.ds(o, 16)] * ws
            return 0

        lax.fori_loop(0, 224, body, 0)

    g_copies = [None] * NCH
    s_copies = [None] * NCH
    g_copies[0] = gather(0)
    for g in range(NCH):
        if g + 1 < NCH:
            if g >= 1:
                s_copies[g - 1].wait()
            g_copies[g + 1] = gather(g + 1)
        g_copies[g].wait()
        bb = bufs[g % 2]
        scale_channel(bb, g)
        s_copies[g] = pltpu.async_copy(
            bb, out_hbm.at[pl.ds(half * OUT_C + jbase + g, 1)],
            s_sems[g % 2])
    s_copies[NCH - 2].wait()
    s_copies[NCH - 1].wait()


def _sc_gather(x3, idx, w):
    mesh = plsc.VectorSubcoreMesh(core_axis_name="c", subcore_axis_name="s")
    f = pl.kernel(
        _sc_body,
        mesh=mesh,
        compiler_params=pltpu.CompilerParams(
            needs_layout_passes=False, use_tc_tiling_on_sc=True),
        out_type=jax.ShapeDtypeStruct((2 * OUT_C, 224, 224), jnp.float32),
        scratch_types=[
            pltpu.VMEM((OUT_C,), jnp.int32),
            pltpu.VMEM((OUT_C,), jnp.float32),
            pltpu.VMEM((LIST_N + 16,), jnp.int32),
            pltpu.VMEM((LIST_N + 16,), jnp.float32),
            pltpu.VMEM((1, 224, 224), jnp.float32),
            pltpu.VMEM((1, 224, 224), jnp.float32),
            pltpu.SemaphoreType.DMA,
            pltpu.SemaphoreType.DMA,
            pltpu.SemaphoreType.DMA,
            pltpu.SemaphoreType.DMA,
        ],
    )
    return f(x3, idx, w)


@jax.jit
def kernel(x, params):
    w, idx = _topk(params)
    x3 = x.reshape(2 * IN_C, 224, 224)
    out = _sc_gather(x3, idx, w)
    return out.reshape(2, OUT_C, 224, 224)


# speedup vs baseline: 3.1421x; 1.0035x over previous
import jax
import jax.numpy as jnp
from jax import lax
from jax.experimental import pallas as pl
from jax.experimental.pallas import tpu as pltpu
from jax.experimental.pallas import tpu_sc as plsc

IN_C = 384
OUT_C = 192
HW = 224 * 224


def _topk_body(p_row_ref, p_col_ref, vals_ref, idx_ref):
    p_row = p_row_ref[...]
    p_col = p_col_ref[...]
    gt = (p_row > p_col).astype(jnp.int32)
    jj = lax.broadcasted_iota(jnp.int32, (IN_C, IN_C), 1)
    ii = lax.broadcasted_iota(jnp.int32, (IN_C, IN_C), 0)
    tie = ((p_row == p_col) & (jj < ii)).astype(jnp.int32)
    rank = jnp.sum(gt + tie, axis=1, keepdims=True)
    rr = lax.broadcasted_iota(jnp.int32, (IN_C, OUT_C), 1)
    m = rank == rr
    vals_ref[...] = jnp.sum(
        jnp.where(m, p_col, jnp.float32(0)), axis=0, keepdims=True)
    ii_c = lax.broadcasted_iota(jnp.int32, (IN_C, OUT_C), 0)
    idx_ref[...] = jnp.sum(
        jnp.where(m, ii_c, 0), axis=0, keepdims=True)


def _topk(params):
    p_row = params.reshape(1, IN_C)
    p_col = params.reshape(IN_C, 1)
    vals, idx = pl.pallas_call(
        _topk_body,
        out_shape=(
            jax.ShapeDtypeStruct((1, OUT_C), jnp.float32),
            jax.ShapeDtypeStruct((1, OUT_C), jnp.int32),
        ),
    )(p_row, p_col)
    return vals.reshape(OUT_C), idx.reshape(OUT_C)


NW = 32
NCH = 2 * OUT_C // NW
LIST_N = NCH * 8


def _sc_body(x_hbm, idx_hbm, w_hbm, out_hbm,
             sel_idx_v, sel_w_v, idx_list_v, w_list_v,
             buf0, buf1, g_sem0, g_sem1, s_sem0, s_sem1):
    wid = lax.axis_index("s") * 2 + lax.axis_index("c")
    half = wid >> 4
    jbase = (wid & 15) * NCH

    pltpu.sync_copy(idx_hbm, sel_idx_v)
    pltpu.sync_copy(w_hbm, sel_w_v)

    lanes = lax.iota(jnp.int32, 16)
    base_v = jnp.full((16,), jbase * 8, jnp.int32)
    c_outc = jnp.full((16,), OUT_C, jnp.int32)
    for k in range(LIST_N // 16):
        rvec = base_v + jnp.full((16,), k * 16, jnp.int32) + lanes
        j = (rvec >> 3) % c_outc
        ch = plsc.load_gather(sel_idx_v, [j])
        idx_list_v[pl.ds(k * 16, 16)] = jnp.full(
            (16,), half * IN_C, jnp.int32) + ch
        w_list_v[pl.ds(k * 16, 16)] = plsc.load_gather(sel_w_v, [j])

    bufs = (buf0, buf1)
    g_sems = (g_sem0, g_sem1)
    s_sems = (s_sem0, s_sem1)

    def gather(g):
        src = idx_list_v[pl.ds(g * 8, 16)][0]
        return pltpu.async_copy(
            x_hbm.at[pl.ds(src, 1)], bufs[g % 2], g_sems[g % 2])

    def scale_channel(bb, g):
        ws = jnp.full((16,), w_list_v[pl.ds(g * 8, 16)][0], jnp.float32)

        def body(h, _):
            for o in range(0, 224, 16):
                bb[0, h, pl.ds(o, 16)] = bb[0, h, pl.ds(o, 16)] * ws
            return 0

        lax.fori_loop(0, 224, body, 0)

    g_copies = [None] * NCH
    s_copies = [None] * NCH
    g_copies[0] = gather(0)
    for g in range(NCH):
        if g + 1 < NCH:
            if g >= 1:
                s_copies[g - 1].wait()
            g_copies[g + 1] = gather(g + 1)
        g_copies[g].wait()
        bb = bufs[g % 2]
        scale_channel(bb, g)
        s_copies[g] = pltpu.async_copy(
            bb, out_hbm.at[pl.ds(half * OUT_C + jbase + g, 1)],
            s_sems[g % 2])
    s_copies[NCH - 2].wait()
    s_copies[NCH - 1].wait()


def _sc_gather(x3, idx, w):
    mesh = plsc.VectorSubcoreMesh(core_axis_name="c", subcore_axis_name="s")
    f = pl.kernel(
        _sc_body,
        mesh=mesh,
        compiler_params=pltpu.CompilerParams(
            needs_layout_passes=False, use_tc_tiling_on_sc=True),
        out_type=jax.ShapeDtypeStruct((2 * OUT_C, 224, 224), jnp.float32),
        scratch_types=[
            pltpu.VMEM((OUT_C,), jnp.int32),
            pltpu.VMEM((OUT_C,), jnp.float32),
            pltpu.VMEM((LIST_N + 16,), jnp.int32),
            pltpu.VMEM((LIST_N + 16,), jnp.float32),
            pltpu.VMEM((1, 224, 224), jnp.float32),
            pltpu.VMEM((1, 224, 224), jnp.float32),
            pltpu.SemaphoreType.DMA,
            pltpu.SemaphoreType.DMA,
            pltpu.SemaphoreType.DMA,
            pltpu.SemaphoreType.DMA,
        ],
    )
    return f(x3, idx, w)


@jax.jit
def kernel(x, params):
    w, idx = _topk(params)
    x3 = x.reshape(2 * IN_C, 224, 224)
    out = _sc_gather(x3, idx, w)
    return out.reshape(2, OUT_C, 224, 224)
